# Initial kernel scaffold; baseline (speedup 1.0000x reference)
import jax
import jax.numpy as jnp
from jax.experimental import pallas as pl


def kernel(mu, Sigma):
    raise NotImplementedError("write your pallas kernel here")


# speedup vs baseline: 1360.8701x; 1360.8701x over previous
import jax
import jax.numpy as jnp
from jax.experimental import pallas as pl

_N = 4096
_BD = 128
_B = 512


def _diag_body(s_ref, d_ref):
    blk = s_ref[...]
    rows = jax.lax.broadcasted_iota(jnp.int32, (_BD, _BD), 0)
    cols = jax.lax.broadcasted_iota(jnp.int32, (_BD, _BD), 1)
    d_ref[0, :] = jnp.sum(jnp.where(rows == cols, blk, 0.0), axis=0)


def _main_body(mu_i_ref, mu_j_ref, d_i_ref, d_j_ref, a_ref, b_ref,
               mud_ref, sd_ref):
    muj = mu_i_ref[0, :]
    muk = mu_j_ref[0, :]
    dj = d_i_ref[0, :]
    dk = d_j_ref[0, :]
    a = a_ref[...]
    bt = b_ref[...].T
    sd = dj[:, None] + dk[None, :] - a - bt
    sd_ref[...] = jnp.sqrt(jnp.maximum(sd, 1e-6))
    mud_ref[...] = muj[:, None] - muk[None, :]


def kernel(mu, Sigma):
    mu2 = mu.reshape(1, _N)
    nd = _N // _BD
    diag = pl.pallas_call(
        _diag_body,
        grid=(nd,),
        in_specs=[pl.BlockSpec((_BD, _BD), lambda i: (i, i))],
        out_specs=pl.BlockSpec((1, _BD), lambda i: (0, i)),
        out_shape=jax.ShapeDtypeStruct((1, _N), jnp.float32),
    )(Sigma)

    nb = _N // _B
    mud, sd = pl.pallas_call(
        _main_body,
        grid=(nb, nb),
        in_specs=[
            pl.BlockSpec((1, _B), lambda i, j: (0, i)),
            pl.BlockSpec((1, _B), lambda i, j: (0, j)),
            pl.BlockSpec((1, _B), lambda i, j: (0, i)),
            pl.BlockSpec((1, _B), lambda i, j: (0, j)),
            pl.BlockSpec((_B, _B), lambda i, j: (i, j)),
            pl.BlockSpec((_B, _B), lambda i, j: (j, i)),
        ],
        out_specs=[
            pl.BlockSpec((_B, _B), lambda i, j: (i, j)),
            pl.BlockSpec((_B, _B), lambda i, j: (i, j)),
        ],
        out_shape=[
            jax.ShapeDtypeStruct((_N, _N), jnp.float32),
            jax.ShapeDtypeStruct((_N, _N), jnp.float32),
        ],
    )(mu2, mu2, diag, diag, Sigma, Sigma)

    return (mud.reshape(_N * _N), sd.reshape(_N * _N))


# speedup vs baseline: 1512.8262x; 1.1117x over previous
import jax
import jax.numpy as jnp
from jax.experimental import pallas as pl

_N = 4096
_BD = 128
_B = 1024


def _diag_body(s_ref, d_ref):
    blk = s_ref[...]
    rows = jax.lax.broadcasted_iota(jnp.int32, (_BD, _BD), 0)
    cols = jax.lax.broadcasted_iota(jnp.int32, (_BD, _BD), 1)
    d_ref[0, :] = jnp.sum(jnp.where(rows == cols, blk, 0.0), axis=0)


def _main_body(mu_i_ref, mu_j_ref, d_i_ref, d_j_ref, a_ref, b_ref,
               mud_ref, sd_ref):
    muj = mu_i_ref[0, :]
    muk = mu_j_ref[0, :]
    dj = d_i_ref[0, :]
    dk = d_j_ref[0, :]
    a = a_ref[...]
    bt = b_ref[...].T
    sd = dj[:, None] + dk[None, :] - a - bt
    sd_ref[...] = jnp.sqrt(jnp.maximum(sd, 1e-6))
    mud_ref[...] = muj[:, None] - muk[None, :]


def kernel(mu, Sigma):
    mu2 = mu.reshape(1, _N)
    nd = _N // _BD
    diag = pl.pallas_call(
        _diag_body,
        grid=(nd,),
        in_specs=[pl.BlockSpec((_BD, _BD), lambda i: (i, i))],
        out_specs=pl.BlockSpec((1, _BD), lambda i: (0, i)),
        out_shape=jax.ShapeDtypeStruct((1, _N), jnp.float32),
    )(Sigma)

    nb = _N // _B
    mud, sd = pl.pallas_call(
        _main_body,
        grid=(nb, nb),
        in_specs=[
            pl.BlockSpec((1, _B), lambda i, j: (0, i)),
            pl.BlockSpec((1, _B), lambda i, j: (0, j)),
            pl.BlockSpec((1, _B), lambda i, j: (0, i)),
            pl.BlockSpec((1, _B), lambda i, j: (0, j)),
            pl.BlockSpec((_B, _B), lambda i, j: (i, j)),
            pl.BlockSpec((_B, _B), lambda i, j: (j, i)),
        ],
        out_specs=[
            pl.BlockSpec((_B, _B), lambda i, j: (i, j)),
            pl.BlockSpec((_B, _B), lambda i, j: (i, j)),
        ],
        out_shape=[
            jax.ShapeDtypeStruct((_N, _N), jnp.float32),
            jax.ShapeDtypeStruct((_N, _N), jnp.float32),
        ],
    )(mu2, mu2, diag, diag, Sigma, Sigma)

    return (mud.reshape(_N * _N), sd.reshape(_N * _N))
